# baseline (speedup 1.0000x reference)
import functools

import jax
import jax.numpy as jnp
from jax import lax
from jax.experimental import pallas as pl
from jax.experimental.pallas import tpu as pltpu
from jax.experimental.pallas import tpu_sc as plsc

B = 16384
CONT = 13
EMB = 64
NCAT = 26
NDENSE = CONT + 1
DOUT = (NDENSE + NCAT) * EMB

NC = 2
NS = 16
NW = NC * NS
ROWS_PER_W = B // NW
R = 8
GROUPS = ROWS_PER_W // R
IDX_PER_BLK = R * NCAT
GCH = 2
IPG = IDX_PER_BLK // GCH
IDX_ROWS = ROWS_PER_W * NCAT // IPG


def _sc_body(idx_hbm, xc_hbm, w_hbm, b_hbm, tab_hbm, out_hbm,
             idx_v, rows_v, xc_v, w_v, b_v, out_v, sg0, sg1, so0, so1):
    wid = lax.axis_index("s") * NC + lax.axis_index("c")
    sem_g = (sg0, sg1)
    sem_o = (so0, so1)

    pltpu.sync_copy(w_hbm, w_v)
    pltpu.sync_copy(b_hbm, b_v)
    pltpu.sync_copy(idx_hbm.at[pl.ds(wid * IDX_ROWS, IDX_ROWS)], idx_v)
    pltpu.sync_copy(xc_hbm.at[pl.ds(wid * ROWS_PER_W, ROWS_PER_W)], xc_v)

    def gather_start(blk, s):
        for i in range(GCH):
            pltpu.async_copy(
                tab_hbm.at[idx_v.at[blk * GCH + i]],
                rows_v.at[s, pl.ds(i * IPG, IPG)],
                sem_g[s])

    def gather_wait(s):
        for i in range(GCH):
            pltpu.make_async_copy(
                tab_hbm.at[idx_v.at[0]],
                rows_v.at[s, pl.ds(i * IPG, IPG)],
                sem_g[s]).wait()

    def out_start(blk, s):
        pltpu.async_copy(
            out_v.at[s],
            out_hbm.at[pl.ds(wid * ROWS_PER_W + blk * R, R)],
            sem_o[s])

    def out_wait(s):
        pltpu.make_async_copy(
            out_v.at[s],
            out_hbm.at[pl.ds(0, R)],
            sem_o[s]).wait()

    def compute(blk, s):
        @plsc.parallel_loop(0, R, unroll=4)
        def row_body(r):
            for q in range(EMB // 16):
                out_v[s, r, pl.ds(q * 16, 16)] = w_v[0, pl.ds(q * 16, 16)]
            xvec = xc_v[blk * R + r, pl.ds(0, 16)]
            for j in range(1, NDENSE):
                sc = xvec[j - 1]
                for q in range(EMB // 16):
                    out_v[s, r, pl.ds(j * EMB + q * 16, 16)] = (
                        w_v[j, pl.ds(q * 16, 16)] * sc
                        + b_v[j - 1, pl.ds(q * 16, 16)])
            for k in range(NCAT):
                for q in range(EMB // 16):
                    out_v[s, r, pl.ds(NDENSE * EMB + k * EMB + q * 16, 16)] = (
                        rows_v[s, r * NCAT + k, pl.ds(q * 16, 16)]
                        + b_v[CONT + k, pl.ds(q * 16, 16)])

    gather_start(0, 0)
    gather_start(1, 1)

    def pair_body(i, carry):
        for s in range(2):
            blk = 2 * i + s
            gather_wait(s)
            pl.when(blk >= 2)(lambda: out_wait(s))
            compute(blk, s)
            out_start(blk, s)
            pl.when(blk + 2 < GROUPS)(lambda: gather_start(blk + 2, s))
        return carry

    lax.fori_loop(0, GROUPS // 2, pair_body, 0)
    out_wait(0)
    out_wait(1)


@functools.partial(
    pl.kernel,
    out_type=jax.ShapeDtypeStruct((B, DOUT), jnp.float32),
    mesh=plsc.VectorSubcoreMesh(core_axis_name="c", subcore_axis_name="s"),
    compiler_params=pltpu.CompilerParams(use_tc_tiling_on_sc=False),
    scratch_types=[
        pltpu.VMEM((IDX_ROWS, IPG), jnp.int32),
        pltpu.VMEM((2, IDX_PER_BLK, EMB), jnp.float32),
        pltpu.VMEM((ROWS_PER_W, 16), jnp.float32),
        pltpu.VMEM((NDENSE, EMB), jnp.float32),
        pltpu.VMEM((CONT + NCAT, EMB), jnp.float32),
        pltpu.VMEM((2, R, DOUT), jnp.float32),
        pltpu.SemaphoreType.DMA,
        pltpu.SemaphoreType.DMA,
        pltpu.SemaphoreType.DMA,
        pltpu.SemaphoreType.DMA,
    ],
)
def _tokenizer_sc(idx_hbm, xc_hbm, w_hbm, b_hbm, tab_hbm, out_hbm,
                  idx_v, rows_v, xc_v, w_v, b_v, out_v, sg0, sg1, so0, so1):
    _sc_body(idx_hbm, xc_hbm, w_hbm, b_hbm, tab_hbm, out_hbm,
             idx_v, rows_v, xc_v, w_v, b_v, out_v, sg0, sg1, so0, so1)


def kernel(x_conts, x_cats, weight, bias, cat_table, category_offsets):
    flat_idx = (x_cats.astype(jnp.int32)
                + category_offsets.astype(jnp.int32)[None, :])
    flat_idx = flat_idx.reshape(B * NCAT // IPG, IPG)
    xc_pad = jnp.zeros((B, 16), jnp.float32).at[:, :CONT].set(x_conts)
    return _tokenizer_sc(flat_idx, xc_pad, weight, bias, cat_table)


# speedup vs baseline: 2.3151x; 2.3151x over previous
import functools

import jax
import jax.numpy as jnp
from jax import lax
from jax.experimental import pallas as pl
from jax.experimental.pallas import tpu as pltpu
from jax.experimental.pallas import tpu_sc as plsc

B = 16384
CONT = 13
EMB = 64
NCAT = 26
NDENSE = CONT + 1
DOUT = (NDENSE + NCAT) * EMB
DCOL = NDENSE * EMB
CCOL = NCAT * EMB

NC = 2
NS = 16
NW = NC * NS
ROWS_PER_W = B // NW
R = 16
GROUPS = ROWS_PER_W // R
IDX_PER_BLK = R * NCAT
GCH = 4
IPG = IDX_PER_BLK // GCH
IDX_ROWS = ROWS_PER_W * NCAT // IPG
NSLOT = 4

BM = 256


def _sc_body(idx_hbm, tab_hbm, out_hbm, idx_v, rows_v, *sems):
    wid = lax.axis_index("s") * NC + lax.axis_index("c")
    sem_g = sems[:NSLOT]
    sem_o = sems[NSLOT:]

    pltpu.sync_copy(idx_hbm.at[pl.ds(wid * IDX_ROWS, IDX_ROWS)], idx_v)

    def gather_start(blk, s):
        for i in range(GCH):
            pltpu.async_copy(
                tab_hbm.at[idx_v.at[blk * GCH + i]],
                rows_v.at[s, pl.ds(i * IPG, IPG)],
                sem_g[s])

    def gather_wait(s):
        for i in range(GCH):
            pltpu.make_async_copy(
                tab_hbm.at[idx_v.at[0]],
                rows_v.at[s, pl.ds(i * IPG, IPG)],
                sem_g[s]).wait()

    def out_start(blk, s):
        pltpu.async_copy(
            rows_v.at[s],
            out_hbm.at[pl.ds(wid * ROWS_PER_W * NCAT + blk * IDX_PER_BLK,
                             IDX_PER_BLK)],
            sem_o[s])

    def out_wait(s):
        pltpu.make_async_copy(
            rows_v.at[s],
            out_hbm.at[pl.ds(0, IDX_PER_BLK)],
            sem_o[s]).wait()

    for b in range(NSLOT):
        gather_start(b, b)

    def ring_body(i, carry):
        for b in range(NSLOT):
            blk = NSLOT * i + b
            gather_wait(b)
            out_start(blk, b)
            s2 = (b + 2) % NSLOT

            def prefetch(blk=blk, s2=s2):
                out_wait(s2)
                gather_start(blk + 2, s2)

            pl.when(jnp.logical_and(blk >= 2, blk + 2 < GROUPS))(prefetch)
        return carry

    lax.fori_loop(0, GROUPS // NSLOT, ring_body, 0)
    for b in range(NSLOT):
        out_wait(b)


@functools.partial(
    pl.kernel,
    out_type=jax.ShapeDtypeStruct((B * NCAT, EMB), jnp.float32),
    mesh=plsc.VectorSubcoreMesh(core_axis_name="c", subcore_axis_name="s"),
    compiler_params=pltpu.CompilerParams(use_tc_tiling_on_sc=False),
    scratch_types=[
        pltpu.VMEM((IDX_ROWS, IPG), jnp.int32),
        pltpu.VMEM((NSLOT, IDX_PER_BLK, EMB), jnp.float32),
    ] + [pltpu.SemaphoreType.DMA] * (2 * NSLOT),
)
def _gather_sc(idx_hbm, tab_hbm, out_hbm, idx_v, rows_v, *sems):
    _sc_body(idx_hbm, tab_hbm, out_hbm, idx_v, rows_v, *sems)


def _assemble_tc(xc_ref, cat_ref, wf_ref, bf_ref, bc_ref, out_ref):
    col = lax.broadcasted_iota(jnp.int32, (16, DCOL), 1) // EMB
    row = lax.broadcasted_iota(jnp.int32, (16, DCOL), 0)
    sel = (col == row).astype(jnp.float32)
    xrep = jnp.dot(xc_ref[...], sel, preferred_element_type=jnp.float32)
    out_ref[:, :DCOL] = xrep * wf_ref[...] + bf_ref[...]
    out_ref[:, DCOL:] = cat_ref[...] + bc_ref[...]


def kernel(x_conts, x_cats, weight, bias, cat_table, category_offsets):
    flat_idx = (x_cats.astype(jnp.int32)
                + category_offsets.astype(jnp.int32)[None, :])
    flat_idx = flat_idx.reshape(B * NCAT // IPG, IPG)
    cat_rows = _gather_sc(flat_idx, cat_table)

    xc_pad = jnp.zeros((B, 16), jnp.float32)
    xc_pad = xc_pad.at[:, 0].set(1.0).at[:, 1:NDENSE].set(x_conts)
    wf = weight.reshape(1, DCOL)
    bf = jnp.concatenate(
        [jnp.zeros((1, EMB), jnp.float32), bias[:CONT].reshape(1, CONT * EMB)],
        axis=1)
    bc = bias[CONT:].reshape(1, CCOL)

    return pl.pallas_call(
        _assemble_tc,
        grid=(B // BM,),
        in_specs=[
            pl.BlockSpec((BM, 16), lambda i: (i, 0)),
            pl.BlockSpec((BM, CCOL), lambda i: (i, 0)),
            pl.BlockSpec((1, DCOL), lambda i: (0, 0)),
            pl.BlockSpec((1, DCOL), lambda i: (0, 0)),
            pl.BlockSpec((1, CCOL), lambda i: (0, 0)),
        ],
        out_specs=pl.BlockSpec((BM, DOUT), lambda i: (i, 0)),
        out_shape=jax.ShapeDtypeStruct((B, DOUT), jnp.float32),
    )(xc_pad, cat_rows.reshape(B, CCOL), wf, bf, bc)
